# baseline (speedup 1.0000x reference)
import functools

import jax
import jax.numpy as jnp
from jax import lax
from jax.experimental import pallas as pl
from jax.experimental.pallas import tpu as pltpu
from jax.experimental.pallas import tpu_sc as plsc


def _argmin_body(z2_ref, c2_ref, z_ref, cb_ref, codes_ref, mind_ref, loss_ref):
    i = pl.program_id(0)
    j = pl.program_id(1)
    nj = pl.num_programs(1)
    kt = cb_ref.shape[0]
    bt = z_ref.shape[0]

    m = lax.dot_general(
        z_ref[...] * -2.0, cb_ref[...],
        dimension_numbers=(((1,), (1,)), ((), ())),
        preferred_element_type=jnp.float32,
    )
    z2 = z2_ref[...]
    ng = kt // 128
    rmin = (z2 + m[:, 0:128]) + c2_ref[:, 0:128]
    garg = jnp.zeros((bt, 128), jnp.float32)
    for g in range(1, ng):
        d = (z2 + m[:, g * 128:(g + 1) * 128]) + c2_ref[:, g * 128:(g + 1) * 128]
        better = d < rmin
        rmin = jnp.where(better, d, rmin)
        garg = jnp.where(better, float(g), garg)
    lmin = jnp.min(rmin, axis=1, keepdims=True)
    lane = lax.broadcasted_iota(jnp.int32, (bt, 128), 1).astype(jnp.float32)
    kf = garg * 128.0 + lane
    larg_f = jnp.min(jnp.where(rmin == lmin, kf, 3.0e38), axis=1, keepdims=True)
    larg = larg_f.astype(jnp.int32) + j * kt

    @pl.when(j == 0)
    def _():
        codes_ref[...] = larg
        mind_ref[...] = lmin

    @pl.when(j > 0)
    def _():
        better = lmin < mind_ref[...]
        codes_ref[...] = jnp.where(better, larg, codes_ref[...])
        mind_ref[...] = jnp.where(better, lmin, mind_ref[...])

    @pl.when(j == nj - 1)
    def _():
        loss_ref[...] = jnp.full((1, 1, 1), 0.0, jnp.float32) + jnp.sum(mind_ref[...])


def _vq_argmin(z, codebook, z2, c2row):
    B, D = z.shape
    K = codebook.shape[0]
    bt = min(1024, B)
    kt = min(8192, K)
    return pl.pallas_call(
        _argmin_body,
        grid=(B // bt, K // kt),
        compiler_params=pltpu.CompilerParams(
            dimension_semantics=("parallel", "arbitrary")),
        in_specs=[
            pl.BlockSpec((bt, 1), lambda i, j: (i, 0)),
            pl.BlockSpec((1, kt), lambda i, j: (0, j)),
            pl.BlockSpec((bt, D), lambda i, j: (i, 0)),
            pl.BlockSpec((kt, D), lambda i, j: (j, 0)),
        ],
        out_specs=[
            pl.BlockSpec((bt, 1), lambda i, j: (i, 0)),
            pl.BlockSpec((bt, 1), lambda i, j: (i, 0)),
            pl.BlockSpec((1, 1, 1), lambda i, j: (i, 0, 0)),
        ],
        out_shape=[
            jax.ShapeDtypeStruct((B, 1), jnp.int32),
            jax.ShapeDtypeStruct((B, 1), jnp.float32),
            jax.ShapeDtypeStruct((B // bt, 1, 1), jnp.float32),
        ],
    )(z2, c2row, z, codebook)


def _sc_gather(codebook, codes):
    B = codes.shape[0]
    K, D = codebook.shape
    info = plsc.get_sparse_core_info()
    nw = info.num_cores * info.num_subcores
    bw = B // nw
    chunk = min(128, bw)
    mesh = plsc.VectorSubcoreMesh(core_axis_name="c", subcore_axis_name="s")

    @functools.partial(
        pl.kernel,
        mesh=mesh,
        out_type=jax.ShapeDtypeStruct((B, D), jnp.float32),
        scratch_types=[
            pltpu.VMEM((bw,), jnp.int32),
            pltpu.VMEM((chunk, D), jnp.float32),
            pltpu.VMEM((chunk, D), jnp.float32),
            pltpu.SemaphoreType.DMA,
            pltpu.SemaphoreType.DMA,
        ],
    )
    def gk(cb_hbm, idx_hbm, out_hbm, idx_v, buf0, buf1, sem0, sem1):
        wid = lax.axis_index("s") * info.num_cores + lax.axis_index("c")
        base = wid * bw
        nch = bw // chunk
        bufs, sems, cps = (buf0, buf1), (sem0, sem1), [None, None]
        pltpu.sync_copy(idx_hbm.at[pl.ds(base, bw)], idx_v)
        cps[0] = pltpu.async_copy(cb_hbm.at[idx_v.at[pl.ds(0, chunk)]],
                                  buf0, sem0)
        for c in range(nch):
            n = c + 1
            if n < nch:
                cps[n % 2] = pltpu.async_copy(
                    cb_hbm.at[idx_v.at[pl.ds(n * chunk, chunk)]],
                    bufs[n % 2], sems[n % 2])
            cps[c % 2].wait()
            pltpu.sync_copy(bufs[c % 2], out_hbm.at[pl.ds(base + c * chunk, chunk)])

    return gk(codebook, codes)


def kernel(z, codebook):
    B, D = z.shape
    z2 = jnp.sum(z * z, axis=-1, keepdims=True)
    c2 = jnp.sum(codebook * codebook, axis=-1)
    codes2d, _mind, loss_sum = _vq_argmin(z, codebook, z2, c2[None, :])
    codes = codes2d[:, 0]
    quantized = _sc_gather(codebook, codes)
    commit_loss = 0.25 * (jnp.sum(loss_sum) / (B * D))
    return quantized, codes, commit_loss


# speedup vs baseline: 1.0270x; 1.0270x over previous
import functools

import jax
import jax.numpy as jnp
from jax import lax
from jax.experimental import pallas as pl
from jax.experimental.pallas import tpu as pltpu
from jax.experimental.pallas import tpu_sc as plsc


def _argmin_body(z2_ref, c2_ref, z_ref, cb_ref, codes_ref, mind_ref, loss_ref):
    i = pl.program_id(0)
    j = pl.program_id(1)
    nj = pl.num_programs(1)
    kt = cb_ref.shape[0]
    bt = z_ref.shape[0]

    zm2 = z_ref[...] * -2.0
    z2 = z2_ref[...]
    mblk = min(1024, kt)
    dn = (((1,), (1,)), ((), ()))
    rmin = None
    garg = jnp.zeros((bt, 128), jnp.float32)
    for b in range(kt // mblk):
        mb = lax.dot_general(zm2, cb_ref[b * mblk:(b + 1) * mblk, :], dn,
                             preferred_element_type=jnp.float32)
        for gg in range(mblk // 128):
            g = b * (mblk // 128) + gg
            lo = g * 128
            d = (z2 + mb[:, gg * 128:(gg + 1) * 128]) + c2_ref[:, lo:lo + 128]
            if rmin is None:
                rmin = d
                continue
            better = d < rmin
            rmin = jnp.where(better, d, rmin)
            garg = jnp.where(better, float(g), garg)
    lmin = jnp.min(rmin, axis=1, keepdims=True)
    lane = lax.broadcasted_iota(jnp.int32, (bt, 128), 1).astype(jnp.float32)
    kf = garg * 128.0 + lane
    larg_f = jnp.min(jnp.where(rmin == lmin, kf, 3.0e38), axis=1, keepdims=True)
    larg = larg_f.astype(jnp.int32) + j * kt

    @pl.when(j == 0)
    def _():
        codes_ref[...] = larg
        mind_ref[...] = lmin

    @pl.when(j > 0)
    def _():
        better = lmin < mind_ref[...]
        codes_ref[...] = jnp.where(better, larg, codes_ref[...])
        mind_ref[...] = jnp.where(better, lmin, mind_ref[...])

    @pl.when(j == nj - 1)
    def _():
        part = jnp.sum(mind_ref[...])
        prev = jnp.where(i == 0, jnp.zeros((1, 1), jnp.float32), loss_ref[...])
        loss_ref[...] = prev + part


def _vq_argmin(z, codebook, z2, c2row):
    B, D = z.shape
    K = codebook.shape[0]
    bt = min(1024, B)
    kt = min(8192, K)
    return pl.pallas_call(
        _argmin_body,
        grid=(B // bt, K // kt),
        in_specs=[
            pl.BlockSpec((bt, 1), lambda i, j: (i, 0)),
            pl.BlockSpec((1, kt), lambda i, j: (0, j)),
            pl.BlockSpec((bt, D), lambda i, j: (i, 0)),
            pl.BlockSpec((kt, D), lambda i, j: (j, 0)),
        ],
        out_specs=[
            pl.BlockSpec((bt, 1), lambda i, j: (i, 0)),
            pl.BlockSpec((bt, 1), lambda i, j: (i, 0)),
            pl.BlockSpec((1, 1), lambda i, j: (0, 0)),
        ],
        out_shape=[
            jax.ShapeDtypeStruct((B, 1), jnp.int32),
            jax.ShapeDtypeStruct((B, 1), jnp.float32),
            jax.ShapeDtypeStruct((1, 1), jnp.float32),
        ],
    )(z2, c2row, z, codebook)


def _sc_gather(codebook, codes):
    B = codes.shape[0]
    K, D = codebook.shape
    info = plsc.get_sparse_core_info()
    nw = info.num_cores * info.num_subcores
    bw = B // nw
    chunk = min(128, bw)
    mesh = plsc.VectorSubcoreMesh(core_axis_name="c", subcore_axis_name="s")

    @functools.partial(
        pl.kernel,
        mesh=mesh,
        out_type=jax.ShapeDtypeStruct((B, D), jnp.float32),
        scratch_types=[
            pltpu.VMEM((bw,), jnp.int32),
            pltpu.VMEM((chunk, D), jnp.float32),
            pltpu.VMEM((chunk, D), jnp.float32),
            pltpu.SemaphoreType.DMA,
            pltpu.SemaphoreType.DMA,
        ],
    )
    def gk(cb_hbm, idx_hbm, out_hbm, idx_v, buf0, buf1, sem0, sem1):
        wid = lax.axis_index("s") * info.num_cores + lax.axis_index("c")
        base = wid * bw
        nch = bw // chunk
        bufs, sems, cps = (buf0, buf1), (sem0, sem1), [None, None]
        pltpu.sync_copy(idx_hbm.at[pl.ds(base, bw)], idx_v)
        cps[0] = pltpu.async_copy(cb_hbm.at[idx_v.at[pl.ds(0, chunk)]],
                                  buf0, sem0)
        for c in range(nch):
            n = c + 1
            if n < nch:
                cps[n % 2] = pltpu.async_copy(
                    cb_hbm.at[idx_v.at[pl.ds(n * chunk, chunk)]],
                    bufs[n % 2], sems[n % 2])
            cps[c % 2].wait()
            pltpu.sync_copy(bufs[c % 2], out_hbm.at[pl.ds(base + c * chunk, chunk)])

    return gk(codebook, codes)


def kernel(z, codebook):
    B, D = z.shape
    z2 = jnp.sum(z * z, axis=-1, keepdims=True)
    c2 = jnp.sum(codebook * codebook, axis=-1)
    codes2d, _mind, loss_sum = _vq_argmin(z, codebook, z2, c2[None, :])
    codes = codes2d[:, 0]
    quantized = _sc_gather(codebook, codes)
    commit_loss = 0.25 * (loss_sum[0, 0] / (B * D))
    return quantized, codes, commit_loss
